# baseline (speedup 1.0000x reference)
import jax
import jax.numpy as jnp
from jax import lax
from jax.experimental import pallas as pl
from jax.experimental.pallas import tpu as pltpu
from jax.experimental.pallas import tpu_sc as plsc

VOCAB = 22
MAX_LEN = 200
AA_DIM = 128
POS_DIM = 64
BLO_DIM = 22
PC_DIM = 7
FUSED = AA_DIM + POS_DIM + BLO_DIM + PC_DIM
ODD_W = FUSED - 128
ENTRIES = MAX_LEN * VOCAB
B, L = 1024, 200
ROWS = B * L

NUM_CORES = 2
NUM_SUBCORES = 16
NW = NUM_CORES * NUM_SUBCORES
R_PER_W = ROWS // NW
VGRP = R_PER_W // 16
CHUNK = 128
NCHUNK = R_PER_W // CHUNK

B_BLK = 8
OUT_BLK = B_BLK * MAX_LEN
N_BLK = B // B_BLK


def _table_body(aa_ref, pos_ref, blo_ref, pc_ref, even_ref, odd_ref):
    aa = aa_ref[...]
    pos = pos_ref[...]
    blo = blo_ref[...]
    pc = pc_ref[...]
    r2d_l = jax.lax.broadcasted_iota(jnp.int32, (ENTRIES, MAX_LEN), 0) // VOCAB
    j2d_l = jax.lax.broadcasted_iota(jnp.int32, (ENTRIES, MAX_LEN), 1)
    oh_l = (r2d_l == j2d_l).astype(jnp.float32)
    r2d_v = jax.lax.broadcasted_iota(jnp.int32, (ENTRIES, VOCAB), 0) % VOCAB
    j2d_v = jax.lax.broadcasted_iota(jnp.int32, (ENTRIES, VOCAB), 1)
    oh_v = (r2d_v == j2d_v).astype(jnp.float32)

    hi = jax.lax.Precision.HIGHEST
    even_ref[...] = jnp.dot(oh_v, aa, precision=hi,
                            preferred_element_type=jnp.float32)
    p_half = jnp.concatenate(
        [pos, jnp.zeros((MAX_LEN, 128 - POS_DIM), jnp.float32)], axis=1)
    q_half = jnp.concatenate(
        [jnp.zeros((VOCAB, POS_DIM), jnp.float32), blo, pc,
         jnp.zeros((VOCAB, 128 - ODD_W), jnp.float32)], axis=1)
    odd_ref[...] = (jnp.dot(oh_l, p_half, precision=hi,
                            preferred_element_type=jnp.float32)
                    + jnp.dot(oh_v, q_half, precision=hi,
                              preferred_element_type=jnp.float32))


def _build_tables(aa_table, pos_table, blo_table, pc_table):
    return pl.pallas_call(
        _table_body,
        out_shape=(jax.ShapeDtypeStruct((ENTRIES, 128), jnp.float32),
                   jax.ShapeDtypeStruct((ENTRIES, 128), jnp.float32)),
    )(aa_table, pos_table, blo_table, pc_table)


def _sc_body(te_hbm, to_hbm, seq_hbm, ie_hbm, io_hbm,
             seq_v, idx_v, be0, be1, bo0, bo1, gsem, wsem):
    wid = lax.axis_index("s") * NUM_CORES + lax.axis_index("c")
    base = wid * R_PER_W

    pltpu.sync_copy(seq_hbm.at[pl.ds(base, R_PER_W)], seq_v)

    def idx_body(g, _):
        i0 = g * 16
        s16 = seq_v[pl.ds(i0, 16)]
        ivec = i0 + lax.iota(jnp.int32, 16)
        lpos = lax.rem(ivec, MAX_LEN)
        idx_v[pl.ds(i0, 16)] = lpos * VOCAB + s16
        return 0

    lax.fori_loop(0, VGRP, idx_body, 0)

    bufs = ((be0, bo0), (be1, bo1))

    def fire(c, slot):
        isl = idx_v.at[pl.ds(c * CHUNK, CHUNK)]
        pltpu.async_copy(te_hbm.at[isl], bufs[slot][0], gsem)
        pltpu.async_copy(to_hbm.at[isl], bufs[slot][1], gsem)

    def wait_pair(slot):
        pltpu.make_async_copy(te_hbm.at[idx_v.at[pl.ds(0, CHUNK)]],
                              bufs[slot][0], gsem).wait()
        pltpu.make_async_copy(to_hbm.at[idx_v.at[pl.ds(0, CHUNK)]],
                              bufs[slot][1], gsem).wait()

    def write(c, slot):
        r0 = base + c * CHUNK
        pltpu.sync_copy(bufs[slot][0], ie_hbm.at[pl.ds(r0, CHUNK)])
        pltpu.sync_copy(bufs[slot][1], io_hbm.at[pl.ds(r0, CHUNK)])

    fire(0, 0)

    def pipe_body(i, _):
        c = 2 * i
        fire(c + 1, 1)
        wait_pair(0)
        write(c, 0)

        @pl.when(c + 2 < NCHUNK)
        def _():
            fire(c + 2, 0)

        wait_pair(1)
        write(c + 1, 1)
        return 0

    lax.fori_loop(0, NCHUNK // 2, pipe_body, 0)


def _sc_gather():
    mesh = plsc.VectorSubcoreMesh(core_axis_name="c", subcore_axis_name="s")
    return pl.kernel(
        _sc_body,
        mesh=mesh,
        compiler_params=pltpu.CompilerParams(needs_layout_passes=False),
        out_type=(jax.ShapeDtypeStruct((ROWS, 128), jnp.float32),
                  jax.ShapeDtypeStruct((ROWS, 128), jnp.float32)),
        scratch_types=[
            pltpu.VMEM((R_PER_W,), jnp.int32),
            pltpu.VMEM((R_PER_W,), jnp.int32),
            pltpu.VMEM((CHUNK, 128), jnp.float32),
            pltpu.VMEM((CHUNK, 128), jnp.float32),
            pltpu.VMEM((CHUNK, 128), jnp.float32),
            pltpu.VMEM((CHUNK, 128), jnp.float32),
            pltpu.SemaphoreType.DMA,
            pltpu.SemaphoreType.DMA,
        ],
    )


def _fold_body(e_ref, o_ref, out_ref):
    merged = jnp.concatenate(
        [e_ref[...], o_ref[...][:, 0:ODD_W]], axis=1)
    out_ref[...] = merged.reshape(B_BLK, MAX_LEN, FUSED)


def _fold(img_e, img_o):
    return pl.pallas_call(
        _fold_body,
        grid=(N_BLK,),
        in_specs=[pl.BlockSpec((OUT_BLK, 128), lambda i: (i, 0)),
                  pl.BlockSpec((OUT_BLK, 128), lambda i: (i, 0))],
        out_specs=pl.BlockSpec((B_BLK, MAX_LEN, FUSED), lambda i: (i, 0, 0)),
        out_shape=jax.ShapeDtypeStruct((B, L, FUSED), jnp.float32),
    )(img_e, img_o)


def kernel(sequences, aa_table, pos_table, blo_table, pc_table):
    te, to = _build_tables(aa_table, pos_table, blo_table, pc_table)
    seq_flat = sequences.reshape(ROWS).astype(jnp.int32)
    img_e, img_o = _sc_gather()(te, to, seq_flat)
    return _fold(img_e, img_o)


# speedup vs baseline: 1.5878x; 1.5878x over previous
import jax
import jax.numpy as jnp
from jax import lax
from jax.experimental import pallas as pl
from jax.experimental.pallas import tpu as pltpu
from jax.experimental.pallas import tpu_sc as plsc

VOCAB = 22
MAX_LEN = 200
AA_DIM = 128
POS_DIM = 64
BLO_DIM = 22
PC_DIM = 7
FUSED = AA_DIM + POS_DIM + BLO_DIM + PC_DIM
ODD_W = FUSED - 128
ENTRIES = MAX_LEN * VOCAB
B, L = 1024, 200
ROWS = B * L

NUM_CORES = 2
NUM_SUBCORES = 16
NW = NUM_CORES * NUM_SUBCORES
R_PER_W = ROWS // NW
VGRP = R_PER_W // 16
CHUNK = 128
NCHUNK = R_PER_W // CHUNK

B_BLK = 8
OUT_BLK = B_BLK * MAX_LEN
N_BLK = B // B_BLK


def _table_body(aa_ref, pos_ref, blo_ref, pc_ref, even_ref, odd_ref):
    aa = aa_ref[...]
    pos = pos_ref[...]
    blo = blo_ref[...]
    pc = pc_ref[...]
    r2d_l = jax.lax.broadcasted_iota(jnp.int32, (ENTRIES, MAX_LEN), 0) // VOCAB
    j2d_l = jax.lax.broadcasted_iota(jnp.int32, (ENTRIES, MAX_LEN), 1)
    oh_l = (r2d_l == j2d_l).astype(jnp.float32)
    r2d_v = jax.lax.broadcasted_iota(jnp.int32, (ENTRIES, VOCAB), 0) % VOCAB
    j2d_v = jax.lax.broadcasted_iota(jnp.int32, (ENTRIES, VOCAB), 1)
    oh_v = (r2d_v == j2d_v).astype(jnp.float32)

    hi = jax.lax.Precision.HIGHEST
    even_ref[...] = jnp.dot(oh_v, aa, precision=hi,
                            preferred_element_type=jnp.float32)
    p_half = jnp.concatenate(
        [pos, jnp.zeros((MAX_LEN, 128 - POS_DIM), jnp.float32)], axis=1)
    q_half = jnp.concatenate(
        [jnp.zeros((VOCAB, POS_DIM), jnp.float32), blo, pc,
         jnp.zeros((VOCAB, 128 - ODD_W), jnp.float32)], axis=1)
    odd_ref[...] = (jnp.dot(oh_l, p_half, precision=hi,
                            preferred_element_type=jnp.float32)
                    + jnp.dot(oh_v, q_half, precision=hi,
                              preferred_element_type=jnp.float32))


def _build_tables(aa_table, pos_table, blo_table, pc_table):
    return pl.pallas_call(
        _table_body,
        out_shape=(jax.ShapeDtypeStruct((ENTRIES, 128), jnp.float32),
                   jax.ShapeDtypeStruct((ENTRIES, 128), jnp.float32)),
    )(aa_table, pos_table, blo_table, pc_table)


def _sc_body(te_hbm, to_hbm, seq_hbm, out_hbm,
             seq_v, idx_v, be0, be1, bo0, bo1, gsem, wsem):
    wid = lax.axis_index("s") * NUM_CORES + lax.axis_index("c")
    base = wid * R_PER_W

    pltpu.sync_copy(seq_hbm.at[pl.ds(base, R_PER_W)], seq_v)

    def idx_body(g, _):
        i0 = g * 16
        s16 = seq_v[pl.ds(i0, 16)]
        ivec = i0 + lax.iota(jnp.int32, 16)
        lpos = lax.rem(ivec, MAX_LEN)
        idx_v[pl.ds(i0, 16)] = lpos * VOCAB + s16
        return 0

    lax.fori_loop(0, VGRP, idx_body, 0)

    bufs = ((be0, bo0), (be1, bo1))

    def fire(c, slot):
        isl = idx_v.at[pl.ds(c * CHUNK, CHUNK)]
        pltpu.async_copy(te_hbm.at[isl], bufs[slot][0], gsem)
        pltpu.async_copy(to_hbm.at[isl], bufs[slot][1], gsem)

    def wait_pair(slot):
        pltpu.make_async_copy(te_hbm.at[idx_v.at[pl.ds(0, CHUNK)]],
                              bufs[slot][0], gsem).wait()
        pltpu.make_async_copy(to_hbm.at[idx_v.at[pl.ds(0, CHUNK)]],
                              bufs[slot][1], gsem).wait()

    def write(c, slot):
        r0 = base + c * CHUNK
        pltpu.sync_copy(bufs[slot][0],
                        out_hbm.at[pl.ds(r0, CHUNK), pl.ds(0, 128)])
        pltpu.sync_copy(bufs[slot][1],
                        out_hbm.at[pl.ds(r0, CHUNK), pl.ds(128, 128)])

    fire(0, 0)

    def pipe_body(i, _):
        c = 2 * i
        fire(c + 1, 1)
        wait_pair(0)
        write(c, 0)

        @pl.when(c + 2 < NCHUNK)
        def _():
            fire(c + 2, 0)

        wait_pair(1)
        write(c + 1, 1)
        return 0

    lax.fori_loop(0, NCHUNK // 2, pipe_body, 0)


def _sc_gather():
    mesh = plsc.VectorSubcoreMesh(core_axis_name="c", subcore_axis_name="s")
    return pl.kernel(
        _sc_body,
        mesh=mesh,
        compiler_params=pltpu.CompilerParams(needs_layout_passes=False),
        out_type=jax.ShapeDtypeStruct((ROWS, 256), jnp.float32),
        scratch_types=[
            pltpu.VMEM((R_PER_W,), jnp.int32),
            pltpu.VMEM((R_PER_W,), jnp.int32),
            pltpu.VMEM((CHUNK, 128), jnp.float32),
            pltpu.VMEM((CHUNK, 128), jnp.float32),
            pltpu.VMEM((CHUNK, 128), jnp.float32),
            pltpu.VMEM((CHUNK, 128), jnp.float32),
            pltpu.SemaphoreType.DMA,
            pltpu.SemaphoreType.DMA,
        ],
    )


def kernel(sequences, aa_table, pos_table, blo_table, pc_table):
    te, to = _build_tables(aa_table, pos_table, blo_table, pc_table)
    seq_flat = sequences.reshape(ROWS).astype(jnp.int32)
    out256 = _sc_gather()(te, to, seq_flat)
    return out256[:, :FUSED].reshape(B, L, FUSED)
